# baseline (speedup 1.0000x reference)
import functools

import jax
import jax.numpy as jnp
from jax import lax
from jax.experimental import pallas as pl
from jax.experimental.pallas import tpu as pltpu
from jax.experimental.pallas import tpu_sc as plsc

_NC = 2
_NS = 16
_NW = _NC * _NS
_L = 16


def _sc_gather_pairs(indexes, table2):
    B = indexes.shape[0]
    _, D2 = table2.shape
    b_per_w = B // _NW
    mesh = plsc.VectorSubcoreMesh(core_axis_name="c", subcore_axis_name="s")

    @functools.partial(
        pl.kernel,
        out_type=jax.ShapeDtypeStruct((B, D2), jnp.float32),
        mesh=mesh,
        scratch_types=[
            pltpu.VMEM((b_per_w,), jnp.int32),
            pltpu.VMEM((b_per_w,), jnp.int32),
            pltpu.VMEM((b_per_w, D2), jnp.float32),
            pltpu.SemaphoreType.DMA,
        ],
    )
    def gather_kernel(idx_hbm, table_hbm, out_hbm, idx_v, idx2_v, rows_v, sem):
        wid = lax.axis_index("s") * _NC + lax.axis_index("c")
        base = wid * b_per_w
        pltpu.sync_copy(idx_hbm.at[pl.ds(base, b_per_w)], idx_v)

        def halve(i, _):
            v = idx_v[pl.ds(i * _L, _L)]
            idx2_v[pl.ds(i * _L, _L)] = lax.shift_right_logical(v, 1)
            return 0

        lax.fori_loop(0, b_per_w // _L, halve, 0)
        pltpu.async_copy(table_hbm.at[idx2_v], rows_v, sem).wait()
        pltpu.sync_copy(rows_v, out_hbm.at[pl.ds(base, b_per_w)])

    return gather_kernel(indexes, table2)


def _tc_project(pairs, idx_col, W0, W1):
    B, D2 = pairs.shape
    D = D2 // 2
    S = W0.shape[0]
    blk = 2048

    def body(x_ref, i_ref, w0_ref, w1_ref, o_ref):
        x = x_ref[...]
        odd = (i_ref[...] & 1) == 1
        emb = jnp.where(odd, x[:, D:], x[:, :D])
        dn = (((1,), (1,)), ((), ()))
        o_ref[0] = lax.dot_general(
            emb, w0_ref[...], dn, preferred_element_type=jnp.float32)
        o_ref[1] = lax.dot_general(
            emb, w1_ref[...], dn, preferred_element_type=jnp.float32)

    return pl.pallas_call(
        body,
        grid=(B // blk,),
        in_specs=[
            pl.BlockSpec((blk, D2), lambda i: (i, 0)),
            pl.BlockSpec((blk, 1), lambda i: (i, 0)),
            pl.BlockSpec((S, D), lambda i: (0, 0)),
            pl.BlockSpec((S, D), lambda i: (0, 0)),
        ],
        out_specs=pl.BlockSpec((2, blk, S), lambda i: (0, i, 0)),
        out_shape=jax.ShapeDtypeStruct((2, B, S), jnp.float32),
    )(pairs, idx_col, W0, W1)


def kernel(indexes, table, W0, W1):
    indexes = indexes.astype(jnp.int32)
    n, d = table.shape
    table2 = table.reshape(n // 2, 2 * d)
    pairs = _sc_gather_pairs(indexes, table2)
    return _tc_project(pairs, indexes.reshape(-1, 1), W0, W1)
